# initial kernel scaffold (unmeasured)
import jax
import jax.numpy as jnp
from jax import lax
from jax.experimental import pallas as pl
from jax.experimental.pallas import tpu as pltpu


def kernel(
    x,
):
    def body(*refs):
        pass

    out_shape = jax.ShapeDtypeStruct(..., jnp.float32)
    return pl.pallas_call(body, out_shape=out_shape)(...)


# baseline (device time: 121674 ns/iter reference)
import jax
import jax.numpy as jnp
from jax import lax
from jax.experimental import pallas as pl
from jax.experimental.pallas import tpu as pltpu

M, N = 1024, 1024
HALF = [512, 256, 128, 64]


def kernel(x):
    x2 = x.reshape(M, N)

    def body(x_ref, out_ref, r0, r1, r2, r3, send_sems, recv_sems):
        mx = lax.axis_index("x")
        my = lax.axis_index("y")
        mz = lax.axis_index("z")
        zhi = mz // 2
        zlo = mz % 2

        bits = [mx, my, zhi, zlo]
        partners = [
            (1 - mx, my, mz),
            (mx, 1 - my, mz),
            (mx, my, jnp.bitwise_xor(mz, 2)),
            (mx, my, jnp.bitwise_xor(mz, 1)),
        ]
        recv_refs = [r0, r1, r2, r3]

        out_ref[:, :] = x_ref[:, :]

        base = jnp.int32(0)
        for s in range(4):
            half = HALF[s]
            bit = bits[s]
            send_base = base + (1 - bit) * half
            keep_base = base + bit * half
            rdma = pltpu.make_async_remote_copy(
                src_ref=out_ref.at[pl.ds(send_base, half)],
                dst_ref=recv_refs[s],
                send_sem=send_sems.at[s],
                recv_sem=recv_sems.at[s],
                device_id=partners[s],
                device_id_type=pl.DeviceIdType.MESH,
            )
            rdma.start()
            rdma.wait()
            out_ref[pl.ds(keep_base, half), :] = (
                out_ref[pl.ds(keep_base, half), :] + recv_refs[s][:, :]
            )
            base = keep_base

        for s in reversed(range(4)):
            seg = HALF[s]
            bit = bits[s]
            rdma = pltpu.make_async_remote_copy(
                src_ref=out_ref.at[pl.ds(base, seg)],
                dst_ref=out_ref.at[pl.ds(base, seg)],
                send_sem=send_sems.at[4 + s],
                recv_sem=recv_sems.at[4 + s],
                device_id=partners[s],
                device_id_type=pl.DeviceIdType.MESH,
            )
            rdma.start()
            rdma.wait()
            base = base - bit * seg

    return pl.pallas_call(
        body,
        out_shape=jax.ShapeDtypeStruct((M, N), jnp.float32),
        in_specs=[pl.BlockSpec(memory_space=pltpu.VMEM)],
        out_specs=pl.BlockSpec(memory_space=pltpu.VMEM),
        scratch_shapes=[
            pltpu.VMEM((512, N), jnp.float32),
            pltpu.VMEM((256, N), jnp.float32),
            pltpu.VMEM((128, N), jnp.float32),
            pltpu.VMEM((64, N), jnp.float32),
            pltpu.SemaphoreType.DMA((8,)),
            pltpu.SemaphoreType.DMA((8,)),
        ],
    )(x2)


# device time: 66983 ns/iter; 1.8165x vs baseline; 1.8165x over previous
import jax
import jax.numpy as jnp
from jax import lax
from jax.experimental import pallas as pl
from jax.experimental.pallas import tpu as pltpu

M, N = 1024, 1024
NCHUNK = 4
CH = M // NCHUNK
HALF = [128, 64, 32, 16]

PERM = [
    (0, 1, 2, 3),
    (1, 0, 3, 2),
    (3, 2, 0, 1),
    (2, 3, 1, 0),
]


def kernel(x):
    x2 = x.reshape(M, N)

    def body(x_ref, out_ref, rb0, rb1, rb2, rb3, send_sems, recv_sems):
        mx = lax.axis_index("x")
        my = lax.axis_index("y")
        mz = lax.axis_index("z")

        bits = [mx, my, mz // 2, mz % 2]
        partners = [
            (1 - mx, my, mz),
            (mx, 1 - my, mz),
            (mx, my, jnp.bitwise_xor(mz, 2)),
            (mx, my, jnp.bitwise_xor(mz, 1)),
        ]
        rbufs = [rb0, rb1, rb2, rb3]

        out_ref[:, :] = x_ref[:, :]

        base = [jnp.int32(c * CH) for c in range(NCHUNK)]

        def mk_rs(c, s):
            d = PERM[c][s]
            half = HALF[s]
            send_base = base[c] + (1 - bits[d]) * half
            return pltpu.make_async_remote_copy(
                src_ref=out_ref.at[pl.ds(send_base, half)],
                dst_ref=rbufs[s].at[c],
                send_sem=send_sems.at[c * 8 + s],
                recv_sem=recv_sems.at[c * 8 + s],
                device_id=partners[d],
                device_id_type=pl.DeviceIdType.MESH,
            )

        def mk_ag(c, s):
            seg = HALF[s]
            return pltpu.make_async_remote_copy(
                src_ref=out_ref.at[pl.ds(base[c], seg)],
                dst_ref=out_ref.at[pl.ds(base[c], seg)],
                send_sem=send_sems.at[c * 8 + 4 + s],
                recv_sem=recv_sems.at[c * 8 + 4 + s],
                device_id=partners[PERM[c][s]],
                device_id_type=pl.DeviceIdType.MESH,
            )

        rs = {}
        ag = {}
        for c in range(NCHUNK):
            rs[c] = mk_rs(c, 0)
            rs[c].start()
        for s in range(4):
            half = HALF[s]
            for c in range(NCHUNK):
                d = PERM[c][s]
                rs[c].wait()
                keep = base[c] + bits[d] * half
                out_ref[pl.ds(keep, half), :] = (
                    out_ref[pl.ds(keep, half), :] + rbufs[s][c, :, :]
                )
                base[c] = keep
                if s < 3:
                    rs[c] = mk_rs(c, s + 1)
                    rs[c].start()
                else:
                    ag[c] = mk_ag(c, 3)
                    ag[c].start()

        for s in range(3, -1, -1):
            for c in range(NCHUNK):
                d = PERM[c][s]
                ag[c].wait()
                base[c] = base[c] - bits[d] * HALF[s]
                if s > 0:
                    ag[c] = mk_ag(c, s - 1)
                    ag[c].start()

    return pl.pallas_call(
        body,
        out_shape=jax.ShapeDtypeStruct((M, N), jnp.float32),
        in_specs=[pl.BlockSpec(memory_space=pltpu.VMEM)],
        out_specs=pl.BlockSpec(memory_space=pltpu.VMEM),
        scratch_shapes=[
            pltpu.VMEM((NCHUNK, HALF[0], N), jnp.float32),
            pltpu.VMEM((NCHUNK, HALF[1], N), jnp.float32),
            pltpu.VMEM((NCHUNK, HALF[2], N), jnp.float32),
            pltpu.VMEM((NCHUNK, HALF[3], N), jnp.float32),
            pltpu.SemaphoreType.DMA((32,)),
            pltpu.SemaphoreType.DMA((32,)),
        ],
    )(x2)


# device time: 65945 ns/iter; 1.8451x vs baseline; 1.0157x over previous
import jax
import jax.numpy as jnp
from jax import lax
from jax.experimental import pallas as pl
from jax.experimental.pallas import tpu as pltpu

M, N = 1024, 1024
NCHUNK = 4
CHUNK_ROWS = [384, 384, 128, 128]
CHUNK_BASE = [0, 384, 768, 896]
HALVES = [[cr // 2, cr // 4, cr // 8, cr // 16] for cr in CHUNK_ROWS]

PERM = [
    (0, 1, 2, 3),
    (1, 0, 3, 2),
    (3, 2, 0, 1),
    (2, 3, 1, 0),
]


def kernel(x):
    x2 = x.reshape(M, N)

    def body(x_ref, out_ref, *scratch):
        rbufs = scratch[:16]
        send_sems, recv_sems = scratch[16], scratch[17]

        mx = lax.axis_index("x")
        my = lax.axis_index("y")
        mz = lax.axis_index("z")

        bits = [mx, my, mz // 2, mz % 2]
        partners = [
            (1 - mx, my, mz),
            (mx, 1 - my, mz),
            (mx, my, jnp.bitwise_xor(mz, 2)),
            (mx, my, jnp.bitwise_xor(mz, 1)),
        ]

        out_ref[:, :] = x_ref[:, :]

        base = [jnp.int32(CHUNK_BASE[c]) for c in range(NCHUNK)]

        def mk_rs(c, s):
            d = PERM[c][s]
            half = HALVES[c][s]
            send_base = base[c] + (1 - bits[d]) * half
            return pltpu.make_async_remote_copy(
                src_ref=out_ref.at[pl.ds(send_base, half)],
                dst_ref=rbufs[c * 4 + s],
                send_sem=send_sems.at[c * 8 + s],
                recv_sem=recv_sems.at[c * 8 + s],
                device_id=partners[d],
                device_id_type=pl.DeviceIdType.MESH,
            )

        def mk_ag(c, s):
            seg = HALVES[c][s]
            return pltpu.make_async_remote_copy(
                src_ref=out_ref.at[pl.ds(base[c], seg)],
                dst_ref=out_ref.at[pl.ds(base[c], seg)],
                send_sem=send_sems.at[c * 8 + 4 + s],
                recv_sem=recv_sems.at[c * 8 + 4 + s],
                device_id=partners[PERM[c][s]],
                device_id_type=pl.DeviceIdType.MESH,
            )

        rs = {}
        ag = {}
        for c in range(NCHUNK):
            rs[c] = mk_rs(c, 0)
            rs[c].start()
        for s in range(4):
            for c in range(NCHUNK):
                d = PERM[c][s]
                half = HALVES[c][s]
                rs[c].wait()
                keep = base[c] + bits[d] * half
                out_ref[pl.ds(keep, half), :] = (
                    out_ref[pl.ds(keep, half), :] + rbufs[c * 4 + s][:, :]
                )
                base[c] = keep
                if s < 3:
                    rs[c] = mk_rs(c, s + 1)
                    rs[c].start()
                else:
                    ag[c] = mk_ag(c, 3)
                    ag[c].start()

        for s in range(3, -1, -1):
            for c in range(NCHUNK):
                d = PERM[c][s]
                ag[c].wait()
                base[c] = base[c] - bits[d] * HALVES[c][s]
                if s > 0:
                    ag[c] = mk_ag(c, s - 1)
                    ag[c].start()

    return pl.pallas_call(
        body,
        out_shape=jax.ShapeDtypeStruct((M, N), jnp.float32),
        in_specs=[pl.BlockSpec(memory_space=pltpu.VMEM)],
        out_specs=pl.BlockSpec(memory_space=pltpu.VMEM),
        scratch_shapes=[
            pltpu.VMEM((HALVES[c][s], N), jnp.float32)
            for c in range(NCHUNK)
            for s in range(4)
        ]
        + [
            pltpu.SemaphoreType.DMA((32,)),
            pltpu.SemaphoreType.DMA((32,)),
        ],
    )(x2)


# device time: 59467 ns/iter; 2.0461x vs baseline; 1.1089x over previous
import jax
import jax.numpy as jnp
from jax import lax
from jax.experimental import pallas as pl
from jax.experimental.pallas import tpu as pltpu

M, N = 1024, 1024
NCHUNK = 4
CHUNK_ROWS = [384, 384, 128, 128]
CHUNK_BASE = [0, 384, 768, 896]
HALVES = [[cr // 2, cr // 4, cr // 8, cr // 16] for cr in CHUNK_ROWS]

PERM = [
    (0, 1, 2, 3),
    (1, 0, 3, 2),
    (3, 2, 0, 1),
    (2, 3, 1, 0),
]


def kernel(x):
    x2 = x.reshape(M, N)

    def body(x_ref, out_ref, *scratch):
        rbufs = scratch[:16]
        send_sems, recv_sems = scratch[16], scratch[17]

        mx = lax.axis_index("x")
        my = lax.axis_index("y")
        mz = lax.axis_index("z")

        bits = [mx, my, mz // 2, mz % 2]
        partners = [
            (1 - mx, my, mz),
            (mx, 1 - my, mz),
            (mx, my, jnp.bitwise_xor(mz, 2)),
            (mx, my, jnp.bitwise_xor(mz, 1)),
        ]

        barrier_sem = pltpu.get_barrier_semaphore()
        for d in range(4):
            pl.semaphore_signal(
                barrier_sem,
                inc=1,
                device_id=partners[d],
                device_id_type=pl.DeviceIdType.MESH,
            )
        pl.semaphore_wait(barrier_sem, 4)

        out_ref[:, :] = x_ref[:, :]

        base = [jnp.int32(CHUNK_BASE[c]) for c in range(NCHUNK)]

        def mk_rs(c, s):
            d = PERM[c][s]
            half = HALVES[c][s]
            send_base = base[c] + (1 - bits[d]) * half
            return pltpu.make_async_remote_copy(
                src_ref=out_ref.at[pl.ds(send_base, half)],
                dst_ref=rbufs[c * 4 + s],
                send_sem=send_sems.at[c * 8 + s],
                recv_sem=recv_sems.at[c * 8 + s],
                device_id=partners[d],
                device_id_type=pl.DeviceIdType.MESH,
            )

        def mk_ag(c, s):
            seg = HALVES[c][s]
            return pltpu.make_async_remote_copy(
                src_ref=out_ref.at[pl.ds(base[c], seg)],
                dst_ref=out_ref.at[pl.ds(base[c], seg)],
                send_sem=send_sems.at[c * 8 + 4 + s],
                recv_sem=recv_sems.at[c * 8 + 4 + s],
                device_id=partners[PERM[c][s]],
                device_id_type=pl.DeviceIdType.MESH,
            )

        rs = {}
        ag = {}
        for c in range(NCHUNK):
            rs[c] = mk_rs(c, 0)
            rs[c].start()
        for s in range(4):
            for c in range(NCHUNK):
                d = PERM[c][s]
                half = HALVES[c][s]
                rs[c].wait()
                keep = base[c] + bits[d] * half
                out_ref[pl.ds(keep, half), :] = (
                    out_ref[pl.ds(keep, half), :] + rbufs[c * 4 + s][:, :]
                )
                base[c] = keep
                if s < 3:
                    rs[c] = mk_rs(c, s + 1)
                    rs[c].start()
                else:
                    ag[c] = mk_ag(c, 3)
                    ag[c].start()

        for s in range(3, -1, -1):
            for c in range(NCHUNK):
                d = PERM[c][s]
                ag[c].wait()
                base[c] = base[c] - bits[d] * HALVES[c][s]
                if s > 0:
                    ag[c] = mk_ag(c, s - 1)
                    ag[c].start()

    return pl.pallas_call(
        body,
        out_shape=jax.ShapeDtypeStruct((M, N), jnp.float32),
        in_specs=[pl.BlockSpec(memory_space=pltpu.VMEM)],
        out_specs=pl.BlockSpec(memory_space=pltpu.VMEM),
        scratch_shapes=[
            pltpu.VMEM((HALVES[c][s], N), jnp.float32)
            for c in range(NCHUNK)
            for s in range(4)
        ]
        + [
            pltpu.SemaphoreType.DMA((32,)),
            pltpu.SemaphoreType.DMA((32,)),
        ],
        compiler_params=pltpu.CompilerParams(collective_id=0),
    )(x2)
